# baseline (speedup 1.0000x reference)
import functools

import jax
import jax.numpy as jnp
from jax import lax
from jax.experimental import pallas as pl
from jax.experimental.pallas import tpu as pltpu
from jax.experimental.pallas import tpu_sc as plsc

NC, NS, LANES = 2, 16, 16
NW = NC * NS

MAGIC = 12582912.0
TPAD = 4160

N_TOTAL = 4 * 4096 * 2048
PER_W = N_TOTAL // NW
CHUNK = 16384
N_CHUNKS = PER_W // CHUNK
VREGS = CHUNK // LANES


def _compute_chunk(xbuf, ybuf, table_v):
    @pl.loop(0, VREGS, unroll=8)
    def _vreg(i):
        xv = xbuf[pl.ds(i * LANES, LANES)]
        t = xv * 256.0
        t = jnp.maximum(t, -32768.0)
        t = jnp.minimum(t, 32767.0)
        xq = (t + MAGIC) - MAGIC
        idxf = jnp.minimum(jnp.maximum(xq, -2048.0), 2048.0) + 2048.0
        idx = idxf.astype(jnp.int32)
        sv = plsc.load_gather(table_v, [idx])
        p = xq * sv
        yq = (p + MAGIC) - MAGIC
        ybuf[pl.ds(i * LANES, LANES)] = yq * (1.0 / 256.0)


def _sc_body(x_hbm, table_hbm, out_hbm,
             table_v, xb0, xb1, yb0, yb1, si0, si1, so0, so1):
    wid = lax.axis_index("s") * NC + lax.axis_index("c")
    pltpu.sync_copy(table_hbm, table_v)
    base = wid * PER_W
    xbufs, ybufs = (xb0, xb1), (yb0, yb1)
    sins, souts = (si0, si1), (so0, so1)

    pltpu.async_copy(x_hbm.at[pl.ds(base, CHUNK)], xb0, si0)
    pltpu.async_copy(x_hbm.at[pl.ds(base + CHUNK, CHUNK)], xb1, si1)

    @pl.loop(0, N_CHUNKS, step=2)
    def _outer(g0):
        for b in range(2):
            c = g0 + b
            off = base + c * CHUNK
            pltpu.make_async_copy(
                x_hbm.at[pl.ds(off, CHUNK)], xbufs[b], sins[b]).wait()

            @pl.when(c >= 2)
            def _():
                pltpu.make_async_copy(
                    ybufs[b], out_hbm.at[pl.ds(off, CHUNK)], souts[b]).wait()

            _compute_chunk(xbufs[b], ybufs[b], table_v)
            pltpu.async_copy(ybufs[b], out_hbm.at[pl.ds(off, CHUNK)], souts[b])

            @pl.when(c + 2 < N_CHUNKS)
            def _():
                pltpu.async_copy(
                    x_hbm.at[pl.ds(off + 2 * CHUNK, CHUNK)], xbufs[b], sins[b])

    for b in range(2):
        pltpu.make_async_copy(
            ybufs[b], out_hbm.at[pl.ds(base, CHUNK)], souts[b]).wait()


def silu_q88_sc(x, table):
    b, s, d = x.shape
    n = b * s * d
    xf = x.reshape(n)
    tf = (table.astype(jnp.float32) * (1.0 / 256.0))
    tf = jnp.pad(tf, (0, TPAD - tf.shape[0]))
    mesh = plsc.VectorSubcoreMesh(core_axis_name="c", subcore_axis_name="s")
    run = pl.kernel(
        _sc_body,
        out_type=jax.ShapeDtypeStruct((n,), jnp.float32),
        mesh=mesh,
        compiler_params=pltpu.CompilerParams(needs_layout_passes=False),
        scratch_types=[
            pltpu.VMEM((TPAD,), jnp.float32),
            pltpu.VMEM((CHUNK,), jnp.float32),
            pltpu.VMEM((CHUNK,), jnp.float32),
            pltpu.VMEM((CHUNK,), jnp.float32),
            pltpu.VMEM((CHUNK,), jnp.float32),
            pltpu.SemaphoreType.DMA,
            pltpu.SemaphoreType.DMA,
            pltpu.SemaphoreType.DMA,
            pltpu.SemaphoreType.DMA,
        ],
    )
    return run(xf, tf).reshape(b, s, d)


def kernel(x, table):
    return silu_q88_sc(x, table)


# speedup vs baseline: 4.1507x; 4.1507x over previous
import functools

import jax
import jax.numpy as jnp
from jax import lax
from jax.experimental import pallas as pl
from jax.experimental.pallas import tpu as pltpu
from jax.experimental.pallas import tpu_sc as plsc

NC, NS, LANES = 2, 16, 16
NW = NC * NS

MAGIC = 12582912.0
TPAD = 4160

N_TOTAL = 4 * 4096 * 2048
PER_W = N_TOTAL // NW
CHUNK = 16384
N_CHUNKS = PER_W // CHUNK
VREGS = CHUNK // LANES


def _compute_chunk(xbuf, ybuf, table_v):
    @plsc.parallel_loop(0, VREGS, unroll=8)
    def _vreg(i):
        xv = xbuf[pl.ds(i * LANES, LANES)]
        t = xv * 256.0
        t = jnp.maximum(t, -32768.0)
        t = jnp.minimum(t, 32767.0)
        xq = (t + MAGIC) - MAGIC
        idxf = jnp.minimum(jnp.maximum(xq, -2048.0), 2048.0) + 2048.0
        idx = idxf.astype(jnp.int32)
        sv = plsc.load_gather(table_v, [idx])
        p = xq * sv
        yq = (p + MAGIC) - MAGIC
        ybuf[pl.ds(i * LANES, LANES)] = yq * (1.0 / 256.0)


def _sc_body(x_hbm, table_hbm, out_hbm,
             table_v, xb0, xb1, yb0, yb1, si0, si1, so0, so1):
    wid = lax.axis_index("s") * NC + lax.axis_index("c")
    pltpu.sync_copy(table_hbm, table_v)
    base = wid * PER_W
    xbufs, ybufs = (xb0, xb1), (yb0, yb1)
    sins, souts = (si0, si1), (so0, so1)

    pltpu.async_copy(x_hbm.at[pl.ds(base, CHUNK)], xb0, si0)
    pltpu.async_copy(x_hbm.at[pl.ds(base + CHUNK, CHUNK)], xb1, si1)

    @pl.loop(0, N_CHUNKS, step=2)
    def _outer(g0):
        for b in range(2):
            c = g0 + b
            off = base + c * CHUNK
            pltpu.make_async_copy(
                x_hbm.at[pl.ds(off, CHUNK)], xbufs[b], sins[b]).wait()

            @pl.when(c >= 2)
            def _():
                pltpu.make_async_copy(
                    ybufs[b], out_hbm.at[pl.ds(off, CHUNK)], souts[b]).wait()

            _compute_chunk(xbufs[b], ybufs[b], table_v)
            pltpu.async_copy(ybufs[b], out_hbm.at[pl.ds(off, CHUNK)], souts[b])

            @pl.when(c + 2 < N_CHUNKS)
            def _():
                pltpu.async_copy(
                    x_hbm.at[pl.ds(off + 2 * CHUNK, CHUNK)], xbufs[b], sins[b])

    for b in range(2):
        pltpu.make_async_copy(
            ybufs[b], out_hbm.at[pl.ds(base, CHUNK)], souts[b]).wait()


def silu_q88_sc(x, table):
    b, s, d = x.shape
    n = b * s * d
    xf = x.reshape(n)
    tf = (table.astype(jnp.float32) * (1.0 / 256.0))
    tf = jnp.pad(tf, (0, TPAD - tf.shape[0]))
    mesh = plsc.VectorSubcoreMesh(core_axis_name="c", subcore_axis_name="s")
    run = pl.kernel(
        _sc_body,
        out_type=jax.ShapeDtypeStruct((n,), jnp.float32),
        mesh=mesh,
        compiler_params=pltpu.CompilerParams(needs_layout_passes=False),
        scratch_types=[
            pltpu.VMEM((TPAD,), jnp.float32),
            pltpu.VMEM((CHUNK,), jnp.float32),
            pltpu.VMEM((CHUNK,), jnp.float32),
            pltpu.VMEM((CHUNK,), jnp.float32),
            pltpu.VMEM((CHUNK,), jnp.float32),
            pltpu.SemaphoreType.DMA,
            pltpu.SemaphoreType.DMA,
            pltpu.SemaphoreType.DMA,
            pltpu.SemaphoreType.DMA,
        ],
    )
    return run(xf, tf).reshape(b, s, d)


def kernel(x, table):
    return silu_q88_sc(x, table)


# speedup vs baseline: 4.2220x; 1.0172x over previous
import functools

import jax
import jax.numpy as jnp
from jax import lax
from jax.experimental import pallas as pl
from jax.experimental.pallas import tpu as pltpu
from jax.experimental.pallas import tpu_sc as plsc

NC, NS, LANES = 2, 16, 16
NW = NC * NS

MAGIC = 12582912.0
MAGIC16 = 49152.0
TPAD = 4160

N_TOTAL = 4 * 4096 * 2048
PER_W = N_TOTAL // NW
CHUNK = 16384
N_CHUNKS = PER_W // CHUNK
VREGS = CHUNK // LANES


def _compute_chunk(xbuf, ybuf, table_v):
    @plsc.parallel_loop(0, VREGS, unroll=16)
    def _vreg(i):
        xv = xbuf[pl.ds(i * LANES, LANES)]
        a = xv * 256.0 + MAGIC
        c = jnp.maximum(jnp.minimum(a, MAGIC + 32767.0), MAGIC - 32768.0)
        xq = c - MAGIC
        e = jnp.maximum(jnp.minimum(c, MAGIC + 2048.0), MAGIC - 2048.0)
        idx = (e - (MAGIC - 2048.0)).astype(jnp.int32)
        sv = plsc.load_gather(table_v, [idx])
        p = xq * sv
        ybuf[pl.ds(i * LANES, LANES)] = (p + MAGIC16) - MAGIC16


def _sc_body(x_hbm, table_hbm, out_hbm,
             table_v, xb0, xb1, yb0, yb1, si0, si1, so0, so1):
    wid = lax.axis_index("s") * NC + lax.axis_index("c")
    pltpu.sync_copy(table_hbm, table_v)
    base = wid * PER_W
    xbufs, ybufs = (xb0, xb1), (yb0, yb1)
    sins, souts = (si0, si1), (so0, so1)

    pltpu.async_copy(x_hbm.at[pl.ds(base, CHUNK)], xb0, si0)
    pltpu.async_copy(x_hbm.at[pl.ds(base + CHUNK, CHUNK)], xb1, si1)

    @pl.loop(0, N_CHUNKS, step=2)
    def _outer(g0):
        for b in range(2):
            c = g0 + b
            off = base + c * CHUNK
            pltpu.make_async_copy(
                x_hbm.at[pl.ds(off, CHUNK)], xbufs[b], sins[b]).wait()

            @pl.when(c >= 2)
            def _():
                pltpu.make_async_copy(
                    ybufs[b], out_hbm.at[pl.ds(off, CHUNK)], souts[b]).wait()

            _compute_chunk(xbufs[b], ybufs[b], table_v)
            pltpu.async_copy(ybufs[b], out_hbm.at[pl.ds(off, CHUNK)], souts[b])

            @pl.when(c + 2 < N_CHUNKS)
            def _():
                pltpu.async_copy(
                    x_hbm.at[pl.ds(off + 2 * CHUNK, CHUNK)], xbufs[b], sins[b])

    for b in range(2):
        pltpu.make_async_copy(
            ybufs[b], out_hbm.at[pl.ds(base, CHUNK)], souts[b]).wait()


def silu_q88_sc(x, table):
    b, s, d = x.shape
    n = b * s * d
    xf = x.reshape(n)
    tf = (table.astype(jnp.float32) * (1.0 / 65536.0))
    tf = jnp.pad(tf, (0, TPAD - tf.shape[0]))
    mesh = plsc.VectorSubcoreMesh(core_axis_name="c", subcore_axis_name="s")
    run = pl.kernel(
        _sc_body,
        out_type=jax.ShapeDtypeStruct((n,), jnp.float32),
        mesh=mesh,
        compiler_params=pltpu.CompilerParams(needs_layout_passes=False),
        scratch_types=[
            pltpu.VMEM((TPAD,), jnp.float32),
            pltpu.VMEM((CHUNK,), jnp.float32),
            pltpu.VMEM((CHUNK,), jnp.float32),
            pltpu.VMEM((CHUNK,), jnp.float32),
            pltpu.VMEM((CHUNK,), jnp.float32),
            pltpu.SemaphoreType.DMA,
            pltpu.SemaphoreType.DMA,
            pltpu.SemaphoreType.DMA,
            pltpu.SemaphoreType.DMA,
        ],
    )
    return run(xf, tf).reshape(b, s, d)


def kernel(x, table):
    return silu_q88_sc(x, table)


# speedup vs baseline: 6.0427x; 1.4312x over previous
import jax
import jax.numpy as jnp
from jax import lax
from jax.experimental import pallas as pl
from jax.experimental.pallas import tpu as pltpu
from jax.experimental.pallas import tpu_sc as plsc

NC, NS, LANES = 2, 16, 16
NW = NC * NS

MAGIC = 12582912.0
MAGIC16 = 49152.0
TPAD = 4160

ROWS = 4 * 4096
COLS = 2048
PER_W_ROWS = ROWS // NW
CHUNK_ROWS = 8
N_CHUNKS = PER_W_ROWS // CHUNK_ROWS
VREGS_PER_ROW = COLS // LANES


def _compute_chunk(xbuf, ybuf, table_v):
    for r in range(CHUNK_ROWS):
        @plsc.parallel_loop(0, VREGS_PER_ROW, unroll=16)
        def _vreg(i):
            xv = xbuf[r, pl.ds(i * LANES, LANES)]
            a = xv * 256.0 + MAGIC
            c = jnp.maximum(jnp.minimum(a, MAGIC + 32767.0), MAGIC - 32768.0)
            xq = c - MAGIC
            e = jnp.maximum(jnp.minimum(c, MAGIC + 2048.0), MAGIC - 2048.0)
            idx = (e - (MAGIC - 2048.0)).astype(jnp.int32)
            sv = plsc.load_gather(table_v, [idx])
            p = xq * sv
            ybuf[r, pl.ds(i * LANES, LANES)] = (p + MAGIC16) - MAGIC16


def _sc_body(x_hbm, table_hbm, out_hbm,
             table_v, xb0, xb1, yb0, yb1, si0, si1, so0, so1):
    wid = lax.axis_index("s") * NC + lax.axis_index("c")
    pltpu.sync_copy(table_hbm, table_v)
    base = wid * PER_W_ROWS
    xbufs, ybufs = (xb0, xb1), (yb0, yb1)
    sins, souts = (si0, si1), (so0, so1)

    pltpu.async_copy(x_hbm.at[pl.ds(base, CHUNK_ROWS), :], xb0, si0)
    pltpu.async_copy(x_hbm.at[pl.ds(base + CHUNK_ROWS, CHUNK_ROWS), :], xb1, si1)

    @pl.loop(0, N_CHUNKS, step=2)
    def _outer(g0):
        for b in range(2):
            c = g0 + b
            off = base + c * CHUNK_ROWS
            pltpu.make_async_copy(
                x_hbm.at[pl.ds(off, CHUNK_ROWS), :], xbufs[b], sins[b]).wait()

            @pl.when(c >= 2)
            def _():
                pltpu.make_async_copy(
                    ybufs[b], out_hbm.at[pl.ds(off, CHUNK_ROWS), :],
                    souts[b]).wait()

            _compute_chunk(xbufs[b], ybufs[b], table_v)
            pltpu.async_copy(ybufs[b], out_hbm.at[pl.ds(off, CHUNK_ROWS), :],
                             souts[b])

            @pl.when(c + 2 < N_CHUNKS)
            def _():
                pltpu.async_copy(
                    x_hbm.at[pl.ds(off + 2 * CHUNK_ROWS, CHUNK_ROWS), :],
                    xbufs[b], sins[b])

    for b in range(2):
        pltpu.make_async_copy(
            ybufs[b], out_hbm.at[pl.ds(base, CHUNK_ROWS), :], souts[b]).wait()


def kernel(x, table):
    b, s, d = x.shape
    x2 = x.reshape(b * s, d)
    tf = (table.astype(jnp.float32) * (1.0 / 65536.0))
    tf = jnp.pad(tf, (0, TPAD - tf.shape[0]))
    mesh = plsc.VectorSubcoreMesh(core_axis_name="c", subcore_axis_name="s")
    run = pl.kernel(
        _sc_body,
        out_type=jax.ShapeDtypeStruct((b * s, d), jnp.float32),
        mesh=mesh,
        compiler_params=pltpu.CompilerParams(
            needs_layout_passes=False,
            use_tc_tiling_on_sc=True,
        ),
        scratch_types=[
            pltpu.VMEM((TPAD,), jnp.float32),
            pltpu.VMEM((CHUNK_ROWS, COLS), jnp.float32),
            pltpu.VMEM((CHUNK_ROWS, COLS), jnp.float32),
            pltpu.VMEM((CHUNK_ROWS, COLS), jnp.float32),
            pltpu.VMEM((CHUNK_ROWS, COLS), jnp.float32),
            pltpu.SemaphoreType.DMA,
            pltpu.SemaphoreType.DMA,
            pltpu.SemaphoreType.DMA,
            pltpu.SemaphoreType.DMA,
        ],
    )
    return run(x2, tf).reshape(b, s, d)


# speedup vs baseline: 8.9650x; 1.4836x over previous
import jax
import jax.numpy as jnp
from jax import lax
from jax.experimental import pallas as pl
from jax.experimental.pallas import tpu as pltpu
from jax.experimental.pallas import tpu_sc as plsc

NC, NS, LANES = 2, 16, 16
NW = NC * NS

MAGIC = 12582912.0
MAGIC16 = 49152.0
TPAD = 4160

ROWS = 4 * 4096
COLS = 2048
PER_W_ROWS = ROWS // NW
CHUNK_ROWS = 8
N_CHUNKS = PER_W_ROWS // CHUNK_ROWS
VREGS_PER_ROW = COLS // LANES


def _compute_chunk(xbuf, ybuf, table_v):
    @plsc.parallel_loop(0, CHUNK_ROWS * VREGS_PER_ROW, unroll=16)
    def _vreg(i):
            r = (i >> 3) & (CHUNK_ROWS - 1)
            col = ((i >> 6) << 7) + ((i & 7) << 4)
            xv = xbuf[r, pl.ds(col, LANES)]
            a = xv * 256.0 + MAGIC
            c = jnp.maximum(jnp.minimum(a, MAGIC + 32767.0), MAGIC - 32768.0)
            xq = c - MAGIC
            e = jnp.maximum(jnp.minimum(c, MAGIC + 2048.0), MAGIC - 2048.0)
            idx = (e - (MAGIC - 2048.0)).astype(jnp.int32)
            sv = plsc.load_gather(table_v, [idx])
            p = xq * sv
            ybuf[r, pl.ds(col, LANES)] = (p + MAGIC16) - MAGIC16


def _sc_body(x_hbm, table_hbm, out_hbm,
             table_v, xb0, xb1, yb0, yb1, si0, si1, so0, so1):
    wid = lax.axis_index("s") * NC + lax.axis_index("c")
    pltpu.sync_copy(table_hbm, table_v)
    base = wid * PER_W_ROWS
    xbufs, ybufs = (xb0, xb1), (yb0, yb1)
    sins, souts = (si0, si1), (so0, so1)

    pltpu.async_copy(x_hbm.at[pl.ds(base, CHUNK_ROWS), :], xb0, si0)
    pltpu.async_copy(x_hbm.at[pl.ds(base + CHUNK_ROWS, CHUNK_ROWS), :], xb1, si1)

    @pl.loop(0, N_CHUNKS, step=2)
    def _outer(g0):
        for b in range(2):
            c = g0 + b
            off = base + c * CHUNK_ROWS
            pltpu.make_async_copy(
                x_hbm.at[pl.ds(off, CHUNK_ROWS), :], xbufs[b], sins[b]).wait()

            @pl.when(c >= 2)
            def _():
                pltpu.make_async_copy(
                    ybufs[b], out_hbm.at[pl.ds(off, CHUNK_ROWS), :],
                    souts[b]).wait()

            _compute_chunk(xbufs[b], ybufs[b], table_v)
            pltpu.async_copy(ybufs[b], out_hbm.at[pl.ds(off, CHUNK_ROWS), :],
                             souts[b])

            @pl.when(c + 2 < N_CHUNKS)
            def _():
                pltpu.async_copy(
                    x_hbm.at[pl.ds(off + 2 * CHUNK_ROWS, CHUNK_ROWS), :],
                    xbufs[b], sins[b])

    for b in range(2):
        pltpu.make_async_copy(
            ybufs[b], out_hbm.at[pl.ds(base, CHUNK_ROWS), :], souts[b]).wait()


def kernel(x, table):
    b, s, d = x.shape
    x2 = x.reshape(b * s, d)
    tf = (table.astype(jnp.float32) * (1.0 / 65536.0))
    tf = jnp.pad(tf, (0, TPAD - tf.shape[0]))
    mesh = plsc.VectorSubcoreMesh(core_axis_name="c", subcore_axis_name="s")
    run = pl.kernel(
        _sc_body,
        out_type=jax.ShapeDtypeStruct((b * s, d), jnp.float32),
        mesh=mesh,
        compiler_params=pltpu.CompilerParams(
            needs_layout_passes=False,
            use_tc_tiling_on_sc=True,
        ),
        scratch_types=[
            pltpu.VMEM((TPAD,), jnp.float32),
            pltpu.VMEM((CHUNK_ROWS, COLS), jnp.float32),
            pltpu.VMEM((CHUNK_ROWS, COLS), jnp.float32),
            pltpu.VMEM((CHUNK_ROWS, COLS), jnp.float32),
            pltpu.VMEM((CHUNK_ROWS, COLS), jnp.float32),
            pltpu.SemaphoreType.DMA,
            pltpu.SemaphoreType.DMA,
            pltpu.SemaphoreType.DMA,
            pltpu.SemaphoreType.DMA,
        ],
    )
    return run(x2, tf).reshape(b, s, d)
